# baseline (speedup 1.0000x reference)
import functools

import jax
import jax.numpy as jnp
from jax import lax
from jax.experimental import pallas as pl
from jax.experimental.pallas import tpu as pltpu
from jax.experimental.pallas import tpu_sc as plsc

_NC = 2
_NS = 16
_NW = _NC * _NS
_SIZES = (72, 128, 72, 128)
_SP_SLOTS = (1, 3)
_NBUF = len(_SIZES)
_STRIDE = sum(_SIZES)
_OFFS = tuple(sum(_SIZES[:b]) for b in range(_NBUF))


def _gather_sc(idx_flat, table):
    n, = idx_flat.shape
    _, d = table.shape
    b_per_w = n // _NW
    n_outer = b_per_w // _STRIDE
    mesh = plsc.VectorSubcoreMesh(core_axis_name="c", subcore_axis_name="s")

    @functools.partial(
        pl.kernel,
        mesh=mesh,
        out_type=jax.ShapeDtypeStruct((n, d), jnp.float32),
        scratch_types=(
            [pltpu.VMEM((b_per_w,), jnp.int32)]
            + [pltpu.VMEM((sz, d), jnp.float32) for sz in _SIZES]
            + [
                pltpu.VMEM_SHARED(
                    (_NS, len(_SP_SLOTS), _SIZES[_SP_SLOTS[0]], d), jnp.float32
                )
            ]
            + [pltpu.SemaphoreType.DMA] * (3 * _NBUF)
        ),
    )
    def k(idx_hbm, table_hbm, out_hbm, idx_v, *rest):
        rows = rest[:_NBUF]
        sp_v = rest[_NBUF]
        sems = rest[_NBUF + 1:]
        gsems = sems[:_NBUF]
        osems = sems[_NBUF:2 * _NBUF]
        csems = sems[2 * _NBUF:]
        cid = lax.axis_index("c")
        sid = lax.axis_index("s")
        wid = sid * _NC + cid
        base = wid * b_per_w
        pltpu.sync_copy(idx_hbm.at[pl.ds(base, b_per_w)], idx_v)

        def g_copy(g, b):
            off = g * _STRIDE + _OFFS[b]
            return pltpu.make_async_copy(
                table_hbm.at[idx_v.at[pl.ds(off, _SIZES[b])]],
                rows[b],
                gsems[b],
            )

        def x_copy(b):
            return pltpu.make_async_copy(
                rows[b],
                sp_v.at[sid, _SP_SLOTS.index(b)],
                csems[b],
            )

        def o_copy(g, b):
            if b in _SP_SLOTS:
                src = sp_v.at[sid, _SP_SLOTS.index(b)]
            else:
                src = rows[b]
            off = base + g * _STRIDE + _OFFS[b]
            return pltpu.make_async_copy(
                src,
                out_hbm.at[pl.ds(off, _SIZES[b])],
                osems[b],
            )

        def emit_slot(g, b, refill):
            g_copy(g, b).wait()
            if b in _SP_SLOTS:
                x_copy(b).start()
                x_copy(b).wait()
                o_copy(g, b).start()
                if refill:
                    g_copy(g + 1, b).start()
                o_copy(g, b).wait()
            else:
                o_copy(g, b).start()
                o_copy(g, b).wait()
                if refill:
                    g_copy(g + 1, b).start()

        for b in range(_NBUF):
            g_copy(0, b).start()

        def outer(g, carry):
            for b in range(_NBUF):
                emit_slot(g, b, True)
            return carry

        lax.fori_loop(0, n_outer - 1, outer, 0)

        for b in range(_NBUF):
            emit_slot(n_outer - 1, b, False)

    return k(idx_flat, table)


def kernel(actions, table):
    b, t = actions.shape
    flat = actions.reshape(b * t).astype(jnp.int32)
    out = _gather_sc(flat, table)
    return out.reshape(b, t, table.shape[1])


# speedup vs baseline: 1.0259x; 1.0259x over previous
import functools

import jax
import jax.numpy as jnp
from jax import lax
from jax.experimental import pallas as pl
from jax.experimental.pallas import tpu as pltpu
from jax.experimental.pallas import tpu_sc as plsc

_NC = 2
_NS = 16
_NW = _NC * _NS
_CHUNK = 128
_NBUF = 4
_SP_SLOTS = (1, 3)


def _gather_sc(idx_flat, table):
    n, = idx_flat.shape
    _, d = table.shape
    b_per_w = n // _NW
    n_chunks = b_per_w // _CHUNK
    n_outer = n_chunks // _NBUF
    mesh = plsc.VectorSubcoreMesh(core_axis_name="c", subcore_axis_name="s")

    @functools.partial(
        pl.kernel,
        mesh=mesh,
        out_type=jax.ShapeDtypeStruct((n, d), jnp.float32),
        scratch_types=(
            [
                pltpu.VMEM((b_per_w,), jnp.int32),
                pltpu.VMEM((_NBUF, _CHUNK, d), jnp.float32),
                pltpu.VMEM_SHARED((_NS, len(_SP_SLOTS), _CHUNK, d), jnp.float32),
            ]
            + [pltpu.SemaphoreType.DMA] * (3 * _NBUF)
        ),
    )
    def k(idx_hbm, table_hbm, out_hbm, idx_v, rows_v, sp_v, *sems):
        gsems = sems[:_NBUF]
        osems = sems[_NBUF:2 * _NBUF]
        csems = sems[2 * _NBUF:]
        cid = lax.axis_index("c")
        sid = lax.axis_index("s")
        wid = sid * _NC + cid
        base = wid * b_per_w
        pltpu.sync_copy(idx_hbm.at[pl.ds(base, b_per_w)], idx_v)

        def g_copy(j, b):
            return pltpu.make_async_copy(
                table_hbm.at[idx_v.at[pl.ds(j * _CHUNK, _CHUNK)]],
                rows_v.at[b],
                gsems[b],
            )

        def x_copy(b):
            return pltpu.make_async_copy(
                rows_v.at[b],
                sp_v.at[sid, _SP_SLOTS.index(b)],
                csems[b],
            )

        def o_copy(j, b):
            if b in _SP_SLOTS:
                src = sp_v.at[sid, _SP_SLOTS.index(b)]
            else:
                src = rows_v.at[b]
            return pltpu.make_async_copy(
                src,
                out_hbm.at[pl.ds(base + j * _CHUNK, _CHUNK)],
                osems[b],
            )

        def emit_slot(j, b, refill):
            g_copy(j, b).wait()
            if b in _SP_SLOTS:
                x_copy(b).start()
                x_copy(b).wait()
                o_copy(j, b).start()
                if refill:
                    g_copy(j + _NBUF, b).start()
                o_copy(j, b).wait()
            else:
                o_copy(j, b).start()
                o_copy(j, b).wait()
                if refill:
                    g_copy(j + _NBUF, b).start()

        for b in range(_NBUF):
            g_copy(b, b).start()

        def outer(g, carry):
            j0 = g * _NBUF
            for b in range(_NBUF):
                emit_slot(j0 + b, b, True)
            return carry

        lax.fori_loop(0, n_outer - 1, outer, 0)

        j0 = (n_outer - 1) * _NBUF
        for b in range(_NBUF):
            emit_slot(j0 + b, b, False)

    return k(idx_flat, table)


def kernel(actions, table):
    b, t = actions.shape
    flat = actions.reshape(b * t).astype(jnp.int32)
    out = _gather_sc(flat, table)
    return out.reshape(b, t, table.shape[1])
